# Initial kernel scaffold; baseline (speedup 1.0000x reference)
import jax
import jax.numpy as jnp
from jax.experimental import pallas as pl


def kernel(x, token_table, pos_table):
    raise NotImplementedError("write your pallas kernel here")


# speedup vs baseline: 1.1100x; 1.1100x over previous
import functools

import jax
import jax.numpy as jnp
from jax import lax
from jax.experimental import pallas as pl
from jax.experimental.pallas import tpu as pltpu
from jax.experimental.pallas import tpu_sc as plsc

BATCH = 4096
SEQ_LEN = 200
EMBED_DIM = 32

NUM_CORES = 2
NUM_SUBCORES = 16
NUM_WORKERS = NUM_CORES * NUM_SUBCORES

CHUNK = 128
ROWS_PER_WORKER = BATCH * SEQ_LEN // NUM_WORKERS
CHUNKS_PER_WORKER = ROWS_PER_WORKER // CHUNK
POS_REP = SEQ_LEN + CHUNK


def _sc_body(x2d_hbm, tok_hbm, posrep_hbm, out_hbm, idx_v, rows_v, pos_v, sem):
    wid = lax.axis_index("s") * NUM_CORES + lax.axis_index("c")
    idx_row0 = wid * CHUNKS_PER_WORKER

    pltpu.sync_copy(x2d_hbm.at[pl.ds(idx_row0, CHUNKS_PER_WORKER)], idx_v)
    pltpu.sync_copy(posrep_hbm, pos_v)

    @pl.loop(0, CHUNKS_PER_WORKER)
    def _chunk(c):
        pltpu.async_copy(tok_hbm.at[idx_v.at[c]], rows_v, sem).wait()

        p = lax.rem(c * CHUNK, SEQ_LEN)

        @pl.loop(0, CHUNK, unroll=8)
        def _row(r):
            pr = p + r
            for h in range(EMBED_DIM // 16):
                vec = pos_v[pr, pl.ds(h * 16, 16)]
                plsc.addupdate(rows_v.at[r, pl.ds(h * 16, 16)], vec)

        out_row0 = (idx_row0 + c) * CHUNK
        pltpu.sync_copy(rows_v, out_hbm.at[pl.ds(out_row0, CHUNK)])


@jax.jit
def _tok_pos_embed(x2d, token_table, posrep):
    mesh = plsc.VectorSubcoreMesh(core_axis_name="c", subcore_axis_name="s")
    kfn = pl.kernel(
        _sc_body,
        out_type=jax.ShapeDtypeStruct((BATCH * SEQ_LEN, EMBED_DIM), jnp.float32),
        mesh=mesh,
        scratch_types=[
            pltpu.VMEM((CHUNKS_PER_WORKER, CHUNK), jnp.int32),
            pltpu.VMEM((CHUNK, EMBED_DIM), jnp.float32),
            pltpu.VMEM((POS_REP, EMBED_DIM), jnp.float32),
            pltpu.SemaphoreType.DMA,
        ],
        compiler_params=pltpu.CompilerParams(use_tc_tiling_on_sc=False),
    )
    return kfn(x2d, token_table, posrep)


def kernel(x, token_table, pos_table):
    x2d = x.astype(jnp.int32).reshape(BATCH * SEQ_LEN // CHUNK, CHUNK)
    posrep = jnp.concatenate(
        [pos_table[:SEQ_LEN], pos_table[:POS_REP - SEQ_LEN]], axis=0
    )
    out = _tok_pos_embed(x2d, token_table, posrep)
    return out.reshape(BATCH, SEQ_LEN, EMBED_DIM)


# speedup vs baseline: 1.3190x; 1.1883x over previous
import jax
import jax.numpy as jnp
from jax import lax
from jax.experimental import pallas as pl
from jax.experimental.pallas import tpu as pltpu
from jax.experimental.pallas import tpu_sc as plsc

BATCH = 4096
SEQ_LEN = 200
EMBED_DIM = 32

NUM_CORES = 2
NUM_SUBCORES = 16
NUM_WORKERS = NUM_CORES * NUM_SUBCORES

CHUNK = 128
NBUF = 4
ROWS_PER_WORKER = BATCH * SEQ_LEN // NUM_WORKERS
CHUNKS_PER_WORKER = ROWS_PER_WORKER // CHUNK
NGROUPS = CHUNKS_PER_WORKER // NBUF
POS_REP = SEQ_LEN + CHUNK


def _sc_body(x2d_hbm, tok_hbm, posrep_hbm, out_hbm, idx_v, pos_v, *bufs_and_sems):
    rows = bufs_and_sems[:NBUF]
    gsem = bufs_and_sems[NBUF:2 * NBUF]
    ssem = bufs_and_sems[2 * NBUF:3 * NBUF]

    wid = lax.axis_index("s") * NUM_CORES + lax.axis_index("c")
    idx_row0 = wid * CHUNKS_PER_WORKER

    pltpu.sync_copy(x2d_hbm.at[pl.ds(idx_row0, CHUNKS_PER_WORKER)], idx_v)
    pltpu.sync_copy(posrep_hbm, pos_v)

    def start_gather(c, b):
        pltpu.async_copy(tok_hbm.at[idx_v.at[c]], rows[b], gsem[b])

    def wait_gather(c, b):
        pltpu.make_async_copy(tok_hbm.at[idx_v.at[c]], rows[b], gsem[b]).wait()

    def start_store(c, b):
        pltpu.async_copy(
            rows[b], out_hbm.at[pl.ds((idx_row0 + c) * CHUNK, CHUNK)], ssem[b])

    def wait_store(c, b):
        pltpu.make_async_copy(
            rows[b], out_hbm.at[pl.ds((idx_row0 + c) * CHUNK, CHUNK)],
            ssem[b]).wait()

    def add_pos(c, b):
        p = lax.rem(c * CHUNK, SEQ_LEN)

        @pl.loop(0, CHUNK, unroll=8)
        def _row(r):
            pr = p + r
            for h in range(EMBED_DIM // 16):
                vec = pos_v[pr, pl.ds(h * 16, 16)]
                plsc.addupdate(rows[b].at[r, pl.ds(h * 16, 16)], vec)

    for b in range(NBUF):
        start_gather(b, b)

    for b in range(NBUF):
        wait_gather(b, b)
        add_pos(b, b)
        start_store(b, b)
        if b > 0:
            wait_store(b - 1, b - 1)
            start_gather(b - 1 + NBUF, b - 1)

    @pl.loop(1, NGROUPS - 1)
    def _group(g):
        for b in range(NBUF):
            c = g * NBUF + b
            wait_gather(c, b)
            add_pos(c, b)
            start_store(c, b)
            bp = (b - 1) % NBUF
            wait_store(c - 1, bp)
            start_gather(c - 1 + NBUF, bp)

    g = NGROUPS - 1
    for b in range(NBUF):
        c = g * NBUF + b
        wait_gather(c, b)
        add_pos(c, b)
        start_store(c, b)
        if b == 0:
            wait_store(c - 1, NBUF - 1)
            start_gather(c - 1 + NBUF, NBUF - 1)

    for b in range(NBUF):
        wait_store(g * NBUF + b, b)


@jax.jit
def _tok_pos_embed(x2d, token_table, posrep):
    mesh = plsc.VectorSubcoreMesh(core_axis_name="c", subcore_axis_name="s")
    kfn = pl.kernel(
        _sc_body,
        out_type=jax.ShapeDtypeStruct((BATCH * SEQ_LEN, EMBED_DIM), jnp.float32),
        mesh=mesh,
        scratch_types=[
            pltpu.VMEM((CHUNKS_PER_WORKER, CHUNK), jnp.int32),
            pltpu.VMEM((POS_REP, EMBED_DIM), jnp.float32),
        ] + [pltpu.VMEM((CHUNK, EMBED_DIM), jnp.float32) for _ in range(NBUF)]
        + [pltpu.SemaphoreType.DMA for _ in range(2 * NBUF)],
        compiler_params=pltpu.CompilerParams(use_tc_tiling_on_sc=False),
    )
    return kfn(x2d, token_table, posrep)


def kernel(x, token_table, pos_table):
    x2d = x.astype(jnp.int32).reshape(BATCH * SEQ_LEN // CHUNK, CHUNK)
    posrep = jnp.concatenate(
        [pos_table[:SEQ_LEN], pos_table[:POS_REP - SEQ_LEN]], axis=0
    )
    out = _tok_pos_embed(x2d, token_table, posrep)
    return out.reshape(BATCH, SEQ_LEN, EMBED_DIM)


# speedup vs baseline: 1.3201x; 1.0008x over previous
import jax
import jax.numpy as jnp
from jax import lax
from jax.experimental import pallas as pl
from jax.experimental.pallas import tpu as pltpu
from jax.experimental.pallas import tpu_sc as plsc

BATCH = 4096
SEQ_LEN = 200
EMBED_DIM = 32

NUM_CORES = 2
NUM_SUBCORES = 16
NUM_WORKERS = NUM_CORES * NUM_SUBCORES

CHUNK = 128
NBUF = 4
ROWS_PER_WORKER = BATCH * SEQ_LEN // NUM_WORKERS
CHUNKS_PER_WORKER = ROWS_PER_WORKER // CHUNK
NGROUPS = CHUNKS_PER_WORKER // NBUF
POS_REP = SEQ_LEN + CHUNK


def _sc_body(x2d_hbm, tok_hbm, posrep_hbm, out_hbm, idx_v, pos_v, *bufs_and_sems):
    rows = bufs_and_sems[:NBUF]
    gsem = bufs_and_sems[NBUF:2 * NBUF]
    ssem = bufs_and_sems[2 * NBUF:3 * NBUF]

    wid = lax.axis_index("s") * NUM_CORES + lax.axis_index("c")
    idx_row0 = wid * CHUNKS_PER_WORKER

    pltpu.sync_copy(x2d_hbm.at[pl.ds(idx_row0, CHUNKS_PER_WORKER)], idx_v)
    pltpu.sync_copy(posrep_hbm, pos_v)

    def start_gather(c, b):
        pltpu.async_copy(tok_hbm.at[idx_v.at[c]], rows[b], gsem[b])

    def wait_gather(c, b):
        pltpu.make_async_copy(tok_hbm.at[idx_v.at[c]], rows[b], gsem[b]).wait()

    def start_store(c, b):
        pltpu.async_copy(
            rows[b], out_hbm.at[pl.ds((idx_row0 + c) * CHUNK, CHUNK)], ssem[b])

    def wait_store(c, b):
        pltpu.make_async_copy(
            rows[b], out_hbm.at[pl.ds((idx_row0 + c) * CHUNK, CHUNK)],
            ssem[b]).wait()

    def add_pos(c, b):
        p = lax.rem(c * CHUNK, SEQ_LEN)

        @pl.loop(0, CHUNK, unroll=8)
        def _row(r):
            pr = p + r
            for h in range(EMBED_DIM // 16):
                vec = pos_v[pr, pl.ds(h * 16, 16)]
                plsc.addupdate(rows[b].at[r, pl.ds(h * 16, 16)], vec)

    for b in range(NBUF):
        start_gather(b, b)

    for b in range(NBUF):
        wait_gather(b, b)
        add_pos(b, b)
        start_store(b, b)
        if b > 0:
            wait_store(b - 1, b - 1)
            start_gather(b - 1 + NBUF, b - 1)

    @pl.loop(1, NGROUPS - 1)
    def _group(g):
        for b in range(NBUF):
            c = g * NBUF + b
            wait_gather(c, b)
            add_pos(c, b)
            start_store(c, b)
            bp = (b - 1) % NBUF
            wait_store(c - 1, bp)
            start_gather(c - 1 + NBUF, bp)

    g = NGROUPS - 1
    for b in range(NBUF):
        c = g * NBUF + b
        wait_gather(c, b)
        add_pos(c, b)
        start_store(c, b)
        if b == 0:
            wait_store(c - 1, NBUF - 1)
            start_gather(c - 1 + NBUF, NBUF - 1)

    for b in range(NBUF):
        wait_store(g * NBUF + b, b)


@jax.jit
def _tok_pos_embed(x2d, token_table, posrep):
    mesh = plsc.VectorSubcoreMesh(core_axis_name="c", subcore_axis_name="s")
    kfn = pl.kernel(
        _sc_body,
        out_type=jax.ShapeDtypeStruct((BATCH * SEQ_LEN, EMBED_DIM), jnp.float32),
        mesh=mesh,
        scratch_types=[
            pltpu.VMEM((CHUNKS_PER_WORKER, CHUNK), jnp.int32),
            pltpu.VMEM((POS_REP, EMBED_DIM), jnp.float32),
        ] + [pltpu.VMEM((CHUNK, EMBED_DIM), jnp.float32) for _ in range(NBUF)]
        + [pltpu.SemaphoreType.DMA for _ in range(2 * NBUF)],
        compiler_params=pltpu.CompilerParams(
            use_tc_tiling_on_sc=False, needs_layout_passes=False),
    )
    return kfn(x2d, token_table, posrep)


def kernel(x, token_table, pos_table):
    x2d = x.astype(jnp.int32).reshape(BATCH * SEQ_LEN // CHUNK, CHUNK)
    posrep = jnp.concatenate(
        [pos_table[:SEQ_LEN], pos_table[:POS_REP - SEQ_LEN]], axis=0
    )
    out = _tok_pos_embed(x2d, token_table, posrep)
    return out.reshape(BATCH, SEQ_LEN, EMBED_DIM)
